# Initial kernel scaffold; baseline (speedup 1.0000x reference)
import jax
import jax.numpy as jnp
from jax.experimental import pallas as pl


def kernel(x, edge_index, W0, b0, W1, b1, W2, b2):
    raise NotImplementedError("write your pallas kernel here")


# speedup vs baseline: 5.2485x; 5.2485x over previous
import functools

import jax
import jax.numpy as jnp
from jax import lax
from jax.experimental import pallas as pl
from jax.experimental.pallas import tpu as pltpu
from jax.experimental.pallas import tpu_sc as plsc

NC = 2
NS = 16
NW = NC * NS
CK = 80


def _make_agg(n, e, k, with_deg):
  chunks = e // (NW * CK)
  assert e == NW * CK * chunks
  stride = (n - 640) // (NS - 1)
  assert stride % 8 == 0 and stride * (NS - 1) + 640 == n
  rows_per_tile = 640
  full = rows_per_tile // CK
  assert full * CK == rows_per_tile

  mesh = plsc.VectorSubcoreMesh(
      core_axis_name="c", subcore_axis_name="s",
      num_cores=NC, num_subcores=NS)

  out_type = [jax.ShapeDtypeStruct((NC, n, k), jnp.float32)]
  scratch = [
      pltpu.VMEM((CK,), jnp.int32),
      pltpu.VMEM((CK,), jnp.int32),
      pltpu.VMEM((CK, k), jnp.float32),
      pltpu.VMEM_SHARED((n, k), jnp.float32),
      pltpu.SemaphoreType.DMA,
  ]
  if with_deg:
    out_type.append(jax.ShapeDtypeStruct((NC, n, 16), jnp.float32))
    scratch += [
        pltpu.VMEM((CK, 16), jnp.float32),
        pltpu.VMEM_SHARED((n, 16), jnp.float32),
    ]

  def body(g_hbm, src_hbm, dst_hbm, *rest):
    if with_deg:
      out_hbm, deg_hbm, src_v, dst_v, rows_v, acc, sem, ones_v, dacc = rest
    else:
      out_hbm, src_v, dst_v, rows_v, acc, sem = rest
    cid = lax.axis_index("c")
    sid = lax.axis_index("s")
    wid = cid * NS + sid
    r0 = sid * stride

    zeros16 = jnp.zeros((16,), jnp.float32)

    def zrow(i, c):
      for j in range(k // 16):
        rows_v[i, pl.ds(j * 16, 16)] = zeros16
      return c
    lax.fori_loop(0, CK, zrow, 0)
    for t in range(full):
      pltpu.sync_copy(rows_v.at[pl.ds(0, CK)], acc.at[pl.ds(r0 + t * CK, CK)])

    if with_deg:
      def zdrow(i, c):
        ones_v[i, pl.ds(0, 16)] = zeros16
        return c
      lax.fori_loop(0, CK, zdrow, 0)
      for t in range(full):
        pltpu.sync_copy(ones_v.at[pl.ds(0, CK)],
                        dacc.at[pl.ds(r0 + t * CK, CK)])

      def orow(i, c):
        ones_v[i, pl.ds(0, 16)] = jnp.full((16,), 1.0, jnp.float32)
        return c
      lax.fori_loop(0, CK, orow, 0)

    plsc.subcore_barrier()

    e0 = wid * (chunks * CK)

    def step(j, c):
      pltpu.sync_copy(src_hbm.at[pl.ds(e0 + j * CK, CK)], src_v)
      pltpu.sync_copy(dst_hbm.at[pl.ds(e0 + j * CK, CK)], dst_v)
      pltpu.async_copy(g_hbm.at[src_v], rows_v, sem).wait()
      pltpu.sync_copy(rows_v, acc.at[dst_v], add=True)
      if with_deg:
        pltpu.sync_copy(ones_v, dacc.at[dst_v], add=True)
      return c
    lax.fori_loop(0, chunks, step, 0)

    plsc.subcore_barrier()

    pltpu.sync_copy(acc.at[pl.ds(r0, rows_per_tile)],
                    out_hbm.at[cid, pl.ds(r0, rows_per_tile)])
    if with_deg:
      pltpu.sync_copy(dacc.at[pl.ds(r0, rows_per_tile)],
                      deg_hbm.at[cid, pl.ds(r0, rows_per_tile)])

  return pl.kernel(body, out_type=out_type, mesh=mesh, scratch_types=scratch,
                   compiler_params=pltpu.CompilerParams(
                       use_tc_tiling_on_sc=False))


_BR = 1000


def _mm_body(x_ref, w_ref, o_ref):
  o_ref[...] = jnp.dot(x_ref[...], w_ref[...],
                       preferred_element_type=jnp.float32)


def _fuse_body(s_ref, d_ref, b_ref, w_ref, o_ref):
  s = s_ref[0] + s_ref[1]
  deg = d_ref[0] + d_ref[1]
  inv = 1.0 / jnp.maximum(deg[:, 0:1], 1.0)
  h = jnp.maximum(s * inv + b_ref[...], 0.0)
  o_ref[...] = jnp.dot(h, w_ref[...], preferred_element_type=jnp.float32)


def _act_body(s_ref, d_ref, b_ref, o_ref):
  s = s_ref[0] + s_ref[1]
  deg = d_ref[0] + d_ref[1]
  inv = 1.0 / jnp.maximum(deg[:, 0:1], 1.0)
  o_ref[...] = jnp.maximum(s * inv + b_ref[...], 0.0)


def _final_body(s_ref, d_ref, b_ref, w_ref, o_ref):
  s = s_ref[0] + s_ref[1]
  deg = d_ref[0] + d_ref[1]
  inv = 1.0 / jnp.maximum(deg[:, 0:1], 1.0)
  o_ref[...] = jnp.dot(s * inv, w_ref[...],
                       preferred_element_type=jnp.float32) + b_ref[...]


def _mm(x, w):
  n, d = x.shape
  ko = w.shape[1]
  return pl.pallas_call(
      _mm_body,
      grid=(n // _BR,),
      in_specs=[
          pl.BlockSpec((_BR, d), lambda i: (i, 0)),
          pl.BlockSpec((d, ko), lambda i: (0, 0)),
      ],
      out_specs=pl.BlockSpec((_BR, ko), lambda i: (i, 0)),
      out_shape=jax.ShapeDtypeStruct((n, ko), jnp.float32),
  )(x, w)


def _fuse(sums, degp, b, w):
  _, n, k = sums.shape
  ko = w.shape[1]
  return pl.pallas_call(
      _fuse_body,
      grid=(n // _BR,),
      in_specs=[
          pl.BlockSpec((NC, _BR, k), lambda i: (0, i, 0)),
          pl.BlockSpec((NC, _BR, 16), lambda i: (0, i, 0)),
          pl.BlockSpec((1, k), lambda i: (0, 0)),
          pl.BlockSpec((k, ko), lambda i: (0, 0)),
      ],
      out_specs=pl.BlockSpec((_BR, ko), lambda i: (i, 0)),
      out_shape=jax.ShapeDtypeStruct((n, ko), jnp.float32),
  )(sums, degp, b, w)


def _act(sums, degp, b):
  _, n, k = sums.shape
  return pl.pallas_call(
      _act_body,
      grid=(n // _BR,),
      in_specs=[
          pl.BlockSpec((NC, _BR, k), lambda i: (0, i, 0)),
          pl.BlockSpec((NC, _BR, 16), lambda i: (0, i, 0)),
          pl.BlockSpec((1, k), lambda i: (0, 0)),
      ],
      out_specs=pl.BlockSpec((_BR, k), lambda i: (i, 0)),
      out_shape=jax.ShapeDtypeStruct((n, k), jnp.float32),
  )(sums, degp, b)


def _final(sums, degp, b, w):
  _, n, k = sums.shape
  ko = w.shape[1]
  return pl.pallas_call(
      _final_body,
      grid=(n // _BR,),
      in_specs=[
          pl.BlockSpec((NC, _BR, k), lambda i: (0, i, 0)),
          pl.BlockSpec((NC, _BR, 16), lambda i: (0, i, 0)),
          pl.BlockSpec((1, ko), lambda i: (0, 0)),
          pl.BlockSpec((k, ko), lambda i: (0, 0)),
      ],
      out_specs=pl.BlockSpec((_BR, ko), lambda i: (i, 0)),
      out_shape=jax.ShapeDtypeStruct((n, ko), jnp.float32),
  )(sums, degp, b, w)


def kernel(x, edge_index, W0, b0, W1, b1, W2, b2):
  n, d = x.shape
  e = edge_index.shape[1]
  h = W0.shape[1]
  c = W2.shape[1]
  chunks = e // (NW * CK)

  src3 = edge_index[0]
  dst3 = edge_index[1]

  del c
  agg_d_deg = _make_agg(n, e, d, with_deg=True)
  agg_h = _make_agg(n, e, h, with_deg=False)

  g0 = _mm(x, W0)
  s0, degp = agg_d_deg(g0, src3, dst3)
  g1 = _fuse(s0, degp, b0.reshape(1, -1), W1)
  (s1,) = agg_h(g1, src3, dst3)
  h2 = _act(s1, degp, b1.reshape(1, -1))
  (s2,) = agg_h(h2, src3, dst3)
  return _final(s2, degp, b2.reshape(1, -1), W2)


# speedup vs baseline: 10.2809x; 1.9588x over previous
import functools

import jax
import jax.numpy as jnp
from jax import lax
from jax.experimental import pallas as pl
from jax.experimental.pallas import tpu as pltpu
from jax.experimental.pallas import tpu_sc as plsc

NC = 2
NS = 16
NW = NC * NS
CK = 80


def _make_agg(n, e, k, with_deg):
  chunks = e // (NW * CK)
  assert e == NW * CK * chunks
  stride = (n - 640) // (NS - 1)
  assert stride % 8 == 0 and stride * (NS - 1) + 640 == n
  rows_per_tile = 640
  full = rows_per_tile // CK
  assert full * CK == rows_per_tile

  mesh = plsc.VectorSubcoreMesh(
      core_axis_name="c", subcore_axis_name="s",
      num_cores=NC, num_subcores=NS)

  out_type = [jax.ShapeDtypeStruct((NC, n, k), jnp.float32)]
  scratch = [
      pltpu.VMEM((CK,), jnp.int32),
      pltpu.VMEM((CK,), jnp.int32),
      pltpu.VMEM((CK,), jnp.int32),
      pltpu.VMEM((CK,), jnp.int32),
      pltpu.VMEM((CK, k), jnp.float32),
      pltpu.VMEM((CK, k), jnp.float32),
      pltpu.VMEM_SHARED((n, k), jnp.float32),
      pltpu.SemaphoreType.DMA,
      pltpu.SemaphoreType.DMA,
      pltpu.SemaphoreType.DMA,
      pltpu.SemaphoreType.DMA,
      pltpu.SemaphoreType.DMA,
      pltpu.SemaphoreType.DMA,
  ]
  if with_deg:
    out_type.append(jax.ShapeDtypeStruct((NC, n, 16), jnp.float32))
    scratch += [
        pltpu.VMEM((CK, 16), jnp.float32),
        pltpu.VMEM_SHARED((n, 16), jnp.float32),
    ]

  def body(g_hbm, src_hbm, dst_hbm, *rest):
    ones_v = dacc = None
    if with_deg:
      (out_hbm, deg_hbm, srcv0, srcv1, dstv0, dstv1, rows0, rows1, acc,
       semi0, semi1, semg0, semg1, sems0, sems1, ones_v, dacc) = rest
    else:
      (out_hbm, srcv0, srcv1, dstv0, dstv1, rows0, rows1, acc,
       semi0, semi1, semg0, semg1, sems0, sems1) = rest
    srcs, dsts = (srcv0, srcv1), (dstv0, dstv1)
    rows = (rows0, rows1)
    semi, semg, sems = (semi0, semi1), (semg0, semg1), (sems0, sems1)
    rows_v = rows0
    cid = lax.axis_index("c")
    sid = lax.axis_index("s")
    wid = cid * NS + sid
    r0 = sid * stride

    zeros16 = jnp.zeros((16,), jnp.float32)

    def zrow(i, c):
      for j in range(k // 16):
        rows_v[i, pl.ds(j * 16, 16)] = zeros16
      return c
    lax.fori_loop(0, CK, zrow, 0)
    for t in range(full):
      pltpu.sync_copy(rows_v.at[pl.ds(0, CK)], acc.at[pl.ds(r0 + t * CK, CK)])

    if with_deg:
      def zdrow(i, c):
        ones_v[i, pl.ds(0, 16)] = zeros16
        return c
      lax.fori_loop(0, CK, zdrow, 0)
      for t in range(full):
        pltpu.sync_copy(ones_v.at[pl.ds(0, CK)],
                        dacc.at[pl.ds(r0 + t * CK, CK)])

      def orow(i, c):
        ones_v[i, pl.ds(0, 16)] = jnp.full((16,), 1.0, jnp.float32)
        return c
      lax.fori_loop(0, CK, orow, 0)

    plsc.subcore_barrier()

    e0 = wid * (chunks * CK)

    def stage(c, u):
      pltpu.async_copy(src_hbm.at[pl.ds(e0 + c * CK, CK)], srcs[u], semi[u])
      pltpu.async_copy(dst_hbm.at[pl.ds(e0 + c * CK, CK)], dsts[u], semi[u])
      pltpu.make_async_copy(
          src_hbm.at[pl.ds(e0 + c * CK, CK)], srcs[u], semi[u]).wait()
      pltpu.make_async_copy(
          dst_hbm.at[pl.ds(e0 + c * CK, CK)], dsts[u], semi[u]).wait()

    def gather_start(u):
      pltpu.async_copy(g_hbm.at[srcs[u]], rows[u], semg[u])

    def gather_wait(u):
      pltpu.make_async_copy(g_hbm.at[srcs[u]], rows[u], semg[u]).wait()

    def scat_start(u):
      pltpu.async_copy(rows[u], acc.at[dsts[u]], sems[u], add=True)
      if with_deg:
        pltpu.async_copy(ones_v, dacc.at[dsts[u]], sems[u], add=True)

    def scat_wait(u):
      pltpu.make_async_copy(rows[u], acc.at[dsts[u]], sems[u]).wait()
      if with_deg:
        pltpu.make_async_copy(ones_v, dacc.at[dsts[u]], sems[u]).wait()

    assert chunks % 2 == 1 and chunks >= 3
    stage(0, 0)
    gather_start(0)
    stage(1, 1)
    gather_start(1)
    gather_wait(0)
    scat_start(0)

    def sbody(j, c):
      a = 2 * j
      scat_wait(0)
      stage(a, 0)
      gather_start(0)
      gather_wait(1)
      scat_start(1)
      scat_wait(1)
      stage(a + 1, 1)
      gather_start(1)
      gather_wait(0)
      scat_start(0)
      return c
    lax.fori_loop(1, chunks // 2, sbody, 0)

    last = chunks - 1
    scat_wait(0)
    stage(last, 0)
    gather_start(0)
    gather_wait(1)
    scat_start(1)
    gather_wait(0)
    scat_wait(1)
    scat_start(0)
    scat_wait(0)

    plsc.subcore_barrier()

    pltpu.sync_copy(acc.at[pl.ds(r0, rows_per_tile)],
                    out_hbm.at[cid, pl.ds(r0, rows_per_tile)])
    if with_deg:
      pltpu.sync_copy(dacc.at[pl.ds(r0, rows_per_tile)],
                      deg_hbm.at[cid, pl.ds(r0, rows_per_tile)])

  return pl.kernel(body, out_type=out_type, mesh=mesh, scratch_types=scratch,
                   compiler_params=pltpu.CompilerParams(
                       use_tc_tiling_on_sc=False))


_BR = 1000


def _mm_body(x_ref, w_ref, o_ref):
  o_ref[...] = jnp.dot(x_ref[...], w_ref[...],
                       preferred_element_type=jnp.float32)


def _fuse_body(s_ref, d_ref, b_ref, w_ref, o_ref):
  s = s_ref[0] + s_ref[1]
  deg = d_ref[0] + d_ref[1]
  inv = 1.0 / jnp.maximum(deg[:, 0:1], 1.0)
  h = jnp.maximum(s * inv + b_ref[...], 0.0)
  o_ref[...] = jnp.dot(h, w_ref[...], preferred_element_type=jnp.float32)


def _act_body(s_ref, d_ref, b_ref, o_ref):
  s = s_ref[0] + s_ref[1]
  deg = d_ref[0] + d_ref[1]
  inv = 1.0 / jnp.maximum(deg[:, 0:1], 1.0)
  o_ref[...] = jnp.maximum(s * inv + b_ref[...], 0.0)


def _final_body(s_ref, d_ref, b_ref, w_ref, o_ref):
  s = s_ref[0] + s_ref[1]
  deg = d_ref[0] + d_ref[1]
  inv = 1.0 / jnp.maximum(deg[:, 0:1], 1.0)
  o_ref[...] = jnp.dot(s * inv, w_ref[...],
                       preferred_element_type=jnp.float32) + b_ref[...]


def _mm(x, w):
  n, d = x.shape
  ko = w.shape[1]
  return pl.pallas_call(
      _mm_body,
      grid=(n // _BR,),
      in_specs=[
          pl.BlockSpec((_BR, d), lambda i: (i, 0)),
          pl.BlockSpec((d, ko), lambda i: (0, 0)),
      ],
      out_specs=pl.BlockSpec((_BR, ko), lambda i: (i, 0)),
      out_shape=jax.ShapeDtypeStruct((n, ko), jnp.float32),
  )(x, w)


def _fuse(sums, degp, b, w):
  _, n, k = sums.shape
  ko = w.shape[1]
  return pl.pallas_call(
      _fuse_body,
      grid=(n // _BR,),
      in_specs=[
          pl.BlockSpec((NC, _BR, k), lambda i: (0, i, 0)),
          pl.BlockSpec((NC, _BR, 16), lambda i: (0, i, 0)),
          pl.BlockSpec((1, k), lambda i: (0, 0)),
          pl.BlockSpec((k, ko), lambda i: (0, 0)),
      ],
      out_specs=pl.BlockSpec((_BR, ko), lambda i: (i, 0)),
      out_shape=jax.ShapeDtypeStruct((n, ko), jnp.float32),
  )(sums, degp, b, w)


def _act(sums, degp, b):
  _, n, k = sums.shape
  return pl.pallas_call(
      _act_body,
      grid=(n // _BR,),
      in_specs=[
          pl.BlockSpec((NC, _BR, k), lambda i: (0, i, 0)),
          pl.BlockSpec((NC, _BR, 16), lambda i: (0, i, 0)),
          pl.BlockSpec((1, k), lambda i: (0, 0)),
      ],
      out_specs=pl.BlockSpec((_BR, k), lambda i: (i, 0)),
      out_shape=jax.ShapeDtypeStruct((n, k), jnp.float32),
  )(sums, degp, b)


def _final(sums, degp, b, w):
  _, n, k = sums.shape
  ko = w.shape[1]
  return pl.pallas_call(
      _final_body,
      grid=(n // _BR,),
      in_specs=[
          pl.BlockSpec((NC, _BR, k), lambda i: (0, i, 0)),
          pl.BlockSpec((NC, _BR, 16), lambda i: (0, i, 0)),
          pl.BlockSpec((1, ko), lambda i: (0, 0)),
          pl.BlockSpec((k, ko), lambda i: (0, 0)),
      ],
      out_specs=pl.BlockSpec((_BR, ko), lambda i: (i, 0)),
      out_shape=jax.ShapeDtypeStruct((n, ko), jnp.float32),
  )(sums, degp, b, w)


def kernel(x, edge_index, W0, b0, W1, b1, W2, b2):
  n, d = x.shape
  e = edge_index.shape[1]
  h = W0.shape[1]
  c = W2.shape[1]
  chunks = e // (NW * CK)

  src3 = edge_index[0]
  dst3 = edge_index[1]

  del c
  agg_d_deg = _make_agg(n, e, d, with_deg=True)
  agg_h = _make_agg(n, e, h, with_deg=False)

  g0 = _mm(x, W0)
  s0, degp = agg_d_deg(g0, src3, dst3)
  g1 = _fuse(s0, degp, b0.reshape(1, -1), W1)
  (s1,) = agg_h(g1, src3, dst3)
  h2 = _act(s1, degp, b1.reshape(1, -1))
  (s2,) = agg_h(h2, src3, dst3)
  return _final(s2, degp, b2.reshape(1, -1), W2)


# speedup vs baseline: 11.7276x; 1.1407x over previous
import functools

import jax
import jax.numpy as jnp
from jax import lax
from jax.experimental import pallas as pl
from jax.experimental.pallas import tpu as pltpu
from jax.experimental.pallas import tpu_sc as plsc

NC = 2
NS = 16
NW = NC * NS
CK = 128


def _make_agg(n, e, k, with_deg):
  nchunks = e // CK
  assert e == nchunks * CK
  cpt = nchunks // NW
  leftover = nchunks - cpt * NW
  assert cpt % 2 == 0 and cpt >= 4 and leftover < NW
  stride = (n - 640) // (NS - 1)
  assert stride % 8 == 0 and stride * (NS - 1) + 640 == n
  rows_per_tile = 640
  full = rows_per_tile // CK
  assert full * CK == rows_per_tile

  mesh = plsc.VectorSubcoreMesh(
      core_axis_name="c", subcore_axis_name="s",
      num_cores=NC, num_subcores=NS)

  out_type = [jax.ShapeDtypeStruct((NC, n, k), jnp.float32)]
  scratch = [
      pltpu.VMEM((CK,), jnp.int32),
      pltpu.VMEM((CK,), jnp.int32),
      pltpu.VMEM((CK,), jnp.int32),
      pltpu.VMEM((CK,), jnp.int32),
      pltpu.VMEM((CK, k), jnp.float32),
      pltpu.VMEM((CK, k), jnp.float32),
      pltpu.VMEM_SHARED((n, k), jnp.float32),
      pltpu.SemaphoreType.DMA,
      pltpu.SemaphoreType.DMA,
      pltpu.SemaphoreType.DMA,
      pltpu.SemaphoreType.DMA,
      pltpu.SemaphoreType.DMA,
      pltpu.SemaphoreType.DMA,
  ]
  if with_deg:
    out_type.append(jax.ShapeDtypeStruct((NC, n, 16), jnp.float32))
    scratch += [
        pltpu.VMEM((CK, 16), jnp.float32),
        pltpu.VMEM_SHARED((n, 16), jnp.float32),
    ]

  def body(g_hbm, src_hbm, dst_hbm, *rest):
    ones_v = dacc = None
    if with_deg:
      (out_hbm, deg_hbm, srcv0, srcv1, dstv0, dstv1, rows0, rows1, acc,
       semi0, semi1, semg0, semg1, sems0, sems1, ones_v, dacc) = rest
    else:
      (out_hbm, srcv0, srcv1, dstv0, dstv1, rows0, rows1, acc,
       semi0, semi1, semg0, semg1, sems0, sems1) = rest
    srcs, dsts = (srcv0, srcv1), (dstv0, dstv1)
    rows = (rows0, rows1)
    semi, semg, sems = (semi0, semi1), (semg0, semg1), (sems0, sems1)
    rows_v = rows0
    cid = lax.axis_index("c")
    sid = lax.axis_index("s")
    wid = cid * NS + sid
    r0 = sid * stride

    zeros16 = jnp.zeros((16,), jnp.float32)

    def zrow(i, c):
      for j in range(k // 16):
        rows_v[i, pl.ds(j * 16, 16)] = zeros16
      return c
    lax.fori_loop(0, CK, zrow, 0)
    for t in range(full):
      pltpu.sync_copy(rows_v.at[pl.ds(0, CK)], acc.at[pl.ds(r0 + t * CK, CK)])

    if with_deg:
      def zdrow(i, c):
        ones_v[i, pl.ds(0, 16)] = zeros16
        return c
      lax.fori_loop(0, CK, zdrow, 0)
      for t in range(full):
        pltpu.sync_copy(ones_v.at[pl.ds(0, CK)],
                        dacc.at[pl.ds(r0 + t * CK, CK)])

      def orow(i, c):
        ones_v[i, pl.ds(0, 16)] = jnp.full((16,), 1.0, jnp.float32)
        return c
      lax.fori_loop(0, CK, orow, 0)

    plsc.subcore_barrier()

    c0 = wid * cpt

    def stage(c, u):
      pltpu.async_copy(src_hbm.at[pl.ds(c * CK, CK)], srcs[u], semi[u])
      pltpu.async_copy(dst_hbm.at[pl.ds(c * CK, CK)], dsts[u], semi[u])
      pltpu.make_async_copy(
          src_hbm.at[pl.ds(c * CK, CK)], srcs[u], semi[u]).wait()
      pltpu.make_async_copy(
          dst_hbm.at[pl.ds(c * CK, CK)], dsts[u], semi[u]).wait()

    def gather_start(u):
      pltpu.async_copy(g_hbm.at[srcs[u]], rows[u], semg[u])

    def gather_wait(u):
      pltpu.make_async_copy(g_hbm.at[srcs[u]], rows[u], semg[u]).wait()

    def scat_start(u):
      pltpu.async_copy(rows[u], acc.at[dsts[u]], sems[u], add=True)
      if with_deg:
        pltpu.async_copy(ones_v, dacc.at[dsts[u]], sems[u], add=True)

    def scat_wait(u):
      pltpu.make_async_copy(rows[u], acc.at[dsts[u]], sems[u]).wait()
      if with_deg:
        pltpu.make_async_copy(ones_v, dacc.at[dsts[u]], sems[u]).wait()

    stage(c0, 0)
    gather_start(0)
    stage(c0 + 1, 1)
    gather_start(1)
    gather_wait(0)
    scat_start(0)

    def sbody(j, c):
      a = c0 + 2 * j
      scat_wait(0)
      stage(a, 0)
      gather_start(0)
      gather_wait(1)
      scat_start(1)
      scat_wait(1)
      stage(a + 1, 1)
      gather_start(1)
      gather_wait(0)
      scat_start(0)
      return c
    lax.fori_loop(1, cpt // 2, sbody, 0)

    scat_wait(0)
    gather_wait(1)
    scat_start(1)
    scat_wait(1)

    @pl.when(wid < leftover)
    def _():
      lc = NW * cpt + wid
      pltpu.sync_copy(src_hbm.at[pl.ds(lc * CK, CK)], srcs[0])
      pltpu.sync_copy(dst_hbm.at[pl.ds(lc * CK, CK)], dsts[0])
      pltpu.async_copy(g_hbm.at[srcs[0]], rows[0], semg[0]).wait()
      pltpu.sync_copy(rows[0], acc.at[dsts[0]], add=True)
      if with_deg:
        pltpu.sync_copy(ones_v, dacc.at[dsts[0]], add=True)

    plsc.subcore_barrier()

    pltpu.sync_copy(acc.at[pl.ds(r0, rows_per_tile)],
                    out_hbm.at[cid, pl.ds(r0, rows_per_tile)])
    if with_deg:
      pltpu.sync_copy(dacc.at[pl.ds(r0, rows_per_tile)],
                      deg_hbm.at[cid, pl.ds(r0, rows_per_tile)])

  return pl.kernel(body, out_type=out_type, mesh=mesh, scratch_types=scratch,
                   compiler_params=pltpu.CompilerParams(
                       use_tc_tiling_on_sc=False))


_BR = 1000


def _mm_body(x_ref, w_ref, o_ref):
  o_ref[...] = jnp.dot(x_ref[...], w_ref[...],
                       preferred_element_type=jnp.float32)


def _fuse_body(s_ref, d_ref, b_ref, w_ref, o_ref):
  s = s_ref[0] + s_ref[1]
  deg = d_ref[0] + d_ref[1]
  inv = 1.0 / jnp.maximum(deg[:, 0:1], 1.0)
  h = jnp.maximum(s * inv + b_ref[...], 0.0)
  o_ref[...] = jnp.dot(h, w_ref[...], preferred_element_type=jnp.float32)


def _act_body(s_ref, d_ref, b_ref, o_ref):
  s = s_ref[0] + s_ref[1]
  deg = d_ref[0] + d_ref[1]
  inv = 1.0 / jnp.maximum(deg[:, 0:1], 1.0)
  o_ref[...] = jnp.maximum(s * inv + b_ref[...], 0.0)


def _final_body(s_ref, d_ref, b_ref, w_ref, o_ref):
  s = s_ref[0] + s_ref[1]
  deg = d_ref[0] + d_ref[1]
  inv = 1.0 / jnp.maximum(deg[:, 0:1], 1.0)
  o_ref[...] = jnp.dot(s * inv, w_ref[...],
                       preferred_element_type=jnp.float32) + b_ref[...]


def _mm(x, w):
  n, d = x.shape
  ko = w.shape[1]
  return pl.pallas_call(
      _mm_body,
      grid=(n // _BR,),
      in_specs=[
          pl.BlockSpec((_BR, d), lambda i: (i, 0)),
          pl.BlockSpec((d, ko), lambda i: (0, 0)),
      ],
      out_specs=pl.BlockSpec((_BR, ko), lambda i: (i, 0)),
      out_shape=jax.ShapeDtypeStruct((n, ko), jnp.float32),
  )(x, w)


def _fuse(sums, degp, b, w):
  _, n, k = sums.shape
  ko = w.shape[1]
  return pl.pallas_call(
      _fuse_body,
      grid=(n // _BR,),
      in_specs=[
          pl.BlockSpec((NC, _BR, k), lambda i: (0, i, 0)),
          pl.BlockSpec((NC, _BR, 16), lambda i: (0, i, 0)),
          pl.BlockSpec((1, k), lambda i: (0, 0)),
          pl.BlockSpec((k, ko), lambda i: (0, 0)),
      ],
      out_specs=pl.BlockSpec((_BR, ko), lambda i: (i, 0)),
      out_shape=jax.ShapeDtypeStruct((n, ko), jnp.float32),
  )(sums, degp, b, w)


def _act(sums, degp, b):
  _, n, k = sums.shape
  return pl.pallas_call(
      _act_body,
      grid=(n // _BR,),
      in_specs=[
          pl.BlockSpec((NC, _BR, k), lambda i: (0, i, 0)),
          pl.BlockSpec((NC, _BR, 16), lambda i: (0, i, 0)),
          pl.BlockSpec((1, k), lambda i: (0, 0)),
      ],
      out_specs=pl.BlockSpec((_BR, k), lambda i: (i, 0)),
      out_shape=jax.ShapeDtypeStruct((n, k), jnp.float32),
  )(sums, degp, b)


def _final(sums, degp, b, w):
  _, n, k = sums.shape
  ko = w.shape[1]
  return pl.pallas_call(
      _final_body,
      grid=(n // _BR,),
      in_specs=[
          pl.BlockSpec((NC, _BR, k), lambda i: (0, i, 0)),
          pl.BlockSpec((NC, _BR, 16), lambda i: (0, i, 0)),
          pl.BlockSpec((1, ko), lambda i: (0, 0)),
          pl.BlockSpec((k, ko), lambda i: (0, 0)),
      ],
      out_specs=pl.BlockSpec((_BR, ko), lambda i: (i, 0)),
      out_shape=jax.ShapeDtypeStruct((n, ko), jnp.float32),
  )(sums, degp, b, w)


def kernel(x, edge_index, W0, b0, W1, b1, W2, b2):
  n, d = x.shape
  e = edge_index.shape[1]
  h = W0.shape[1]
  c = W2.shape[1]
  chunks = e // (NW * CK)

  src3 = edge_index[0]
  dst3 = edge_index[1]

  del c
  agg_d_deg = _make_agg(n, e, d, with_deg=True)
  agg_h = _make_agg(n, e, h, with_deg=False)

  g0 = _mm(x, W0)
  s0, degp = agg_d_deg(g0, src3, dst3)
  g1 = _fuse(s0, degp, b0.reshape(1, -1), W1)
  (s1,) = agg_h(g1, src3, dst3)
  h2 = _act(s1, degp, b1.reshape(1, -1))
  (s2,) = agg_h(h2, src3, dst3)
  return _final(s2, degp, b2.reshape(1, -1), W2)


# speedup vs baseline: 11.9604x; 1.0199x over previous
import functools

import jax
import jax.numpy as jnp
from jax import lax
from jax.experimental import pallas as pl
from jax.experimental.pallas import tpu as pltpu
from jax.experimental.pallas import tpu_sc as plsc

NC = 2
NS = 16
NW = NC * NS
CK = 128


def _make_agg(n, e, k, with_deg):
  nchunks = e // CK
  assert e == nchunks * CK
  cpt = nchunks // NW
  leftover = nchunks - cpt * NW
  assert cpt % 2 == 0 and cpt >= 4 and leftover < NW
  stride = (n - 640) // (NS - 1)
  assert stride % 8 == 0 and stride * (NS - 1) + 640 == n
  rows_per_tile = 640
  full = rows_per_tile // CK
  assert full * CK == rows_per_tile

  mesh = plsc.VectorSubcoreMesh(
      core_axis_name="c", subcore_axis_name="s",
      num_cores=NC, num_subcores=NS)

  out_type = [jax.ShapeDtypeStruct((NC, n, k), jnp.float32)]
  scratch = [
      pltpu.VMEM((CK,), jnp.int32),
      pltpu.VMEM((CK,), jnp.int32),
      pltpu.VMEM((CK,), jnp.int32),
      pltpu.VMEM((CK,), jnp.int32),
      pltpu.VMEM((CK, k), jnp.float32),
      pltpu.VMEM((CK, k), jnp.float32),
      pltpu.VMEM_SHARED((n, k), jnp.float32),
      pltpu.SemaphoreType.DMA,
      pltpu.SemaphoreType.DMA,
      pltpu.SemaphoreType.DMA,
      pltpu.SemaphoreType.DMA,
      pltpu.SemaphoreType.DMA,
      pltpu.SemaphoreType.DMA,
  ]
  if with_deg:
    out_type.append(jax.ShapeDtypeStruct((NC, n, 16), jnp.float32))
    scratch += [
        pltpu.VMEM((CK, 16), jnp.float32),
        pltpu.VMEM_SHARED((n, 16), jnp.float32),
    ]

  def body(g_hbm, src_hbm, dst_hbm, *rest):
    ones_v = dacc = None
    if with_deg:
      (out_hbm, deg_hbm, srcv0, srcv1, dstv0, dstv1, rows0, rows1, acc,
       semi0, semi1, semg0, semg1, sems0, sems1, ones_v, dacc) = rest
    else:
      (out_hbm, srcv0, srcv1, dstv0, dstv1, rows0, rows1, acc,
       semi0, semi1, semg0, semg1, sems0, sems1) = rest
    srcs, dsts = (srcv0, srcv1), (dstv0, dstv1)
    rows = (rows0, rows1)
    semi, semg, sems = (semi0, semi1), (semg0, semg1), (sems0, sems1)
    rows_v = rows0
    cid = lax.axis_index("c")
    sid = lax.axis_index("s")
    wid = cid * NS + sid
    r0 = sid * stride

    zeros16 = jnp.zeros((16,), jnp.float32)

    def zrow(i, c):
      for j in range(k // 16):
        rows_v[i, pl.ds(j * 16, 16)] = zeros16
      return c
    lax.fori_loop(0, CK, zrow, 0)
    for t in range(full):
      pltpu.sync_copy(rows_v.at[pl.ds(0, CK)], acc.at[pl.ds(r0 + t * CK, CK)])

    if with_deg:
      def zdrow(i, c):
        ones_v[i, pl.ds(0, 16)] = zeros16
        return c
      lax.fori_loop(0, CK, zdrow, 0)
      for t in range(full):
        pltpu.sync_copy(ones_v.at[pl.ds(0, CK)],
                        dacc.at[pl.ds(r0 + t * CK, CK)])

      def orow(i, c):
        ones_v[i, pl.ds(0, 16)] = jnp.full((16,), 1.0, jnp.float32)
        return c
      lax.fori_loop(0, CK, orow, 0)

    plsc.subcore_barrier()

    c0 = wid * cpt

    def stage(c, u):
      pltpu.async_copy(src_hbm.at[pl.ds(c * CK, CK)], srcs[u], semi[u])
      pltpu.async_copy(dst_hbm.at[pl.ds(c * CK, CK)], dsts[u], semi[u])
      pltpu.make_async_copy(
          src_hbm.at[pl.ds(c * CK, CK)], srcs[u], semi[u]).wait()
      pltpu.make_async_copy(
          dst_hbm.at[pl.ds(c * CK, CK)], dsts[u], semi[u]).wait()

    def gather_start(u):
      pltpu.async_copy(g_hbm.at[srcs[u]], rows[u], semg[u])

    def gather_wait(u):
      pltpu.make_async_copy(g_hbm.at[srcs[u]], rows[u], semg[u]).wait()

    def scat_start(u):
      pltpu.async_copy(rows[u], acc.at[dsts[u]], sems[u], add=True)
      if with_deg:
        pltpu.async_copy(ones_v, dacc.at[dsts[u]], sems[u], add=True)

    def scat_wait(u):
      pltpu.make_async_copy(rows[u], acc.at[dsts[u]], sems[u]).wait()
      if with_deg:
        pltpu.make_async_copy(ones_v, dacc.at[dsts[u]], sems[u]).wait()

    stage(c0, 0)
    gather_start(0)
    stage(c0 + 1, 1)
    gather_start(1)
    gather_wait(0)
    scat_start(0)

    def sbody(j, c):
      a = c0 + 2 * j
      scat_wait(0)
      stage(a, 0)
      gather_start(0)
      gather_wait(1)
      scat_start(1)
      scat_wait(1)
      stage(a + 1, 1)
      gather_start(1)
      gather_wait(0)
      scat_start(0)
      return c
    lax.fori_loop(1, cpt // 2, sbody, 0)

    scat_wait(0)
    gather_wait(1)
    scat_start(1)
    scat_wait(1)

    @pl.when(wid < leftover)
    def _():
      lc = NW * cpt + wid
      pltpu.sync_copy(src_hbm.at[pl.ds(lc * CK, CK)], srcs[0])
      pltpu.sync_copy(dst_hbm.at[pl.ds(lc * CK, CK)], dsts[0])
      pltpu.async_copy(g_hbm.at[srcs[0]], rows[0], semg[0]).wait()
      pltpu.sync_copy(rows[0], acc.at[dsts[0]], add=True)
      if with_deg:
        pltpu.sync_copy(ones_v, dacc.at[dsts[0]], add=True)

    plsc.subcore_barrier()

    pltpu.sync_copy(acc.at[pl.ds(r0, rows_per_tile)],
                    out_hbm.at[cid, pl.ds(r0, rows_per_tile)])
    if with_deg:
      pltpu.sync_copy(dacc.at[pl.ds(r0, rows_per_tile)],
                      deg_hbm.at[cid, pl.ds(r0, rows_per_tile)])

  return pl.kernel(body, out_type=out_type, mesh=mesh, scratch_types=scratch,
                   compiler_params=pltpu.CompilerParams(
                       use_tc_tiling_on_sc=False))


_BR = 2000


def _mm_body(x_ref, w_ref, o_ref):
  o_ref[...] = jnp.dot(x_ref[...], w_ref[...],
                       preferred_element_type=jnp.float32)


def _fuse_body(s_ref, d_ref, b_ref, w_ref, o_ref):
  s = s_ref[0] + s_ref[1]
  deg = d_ref[0] + d_ref[1]
  inv = 1.0 / jnp.maximum(deg[:, 0:1], 1.0)
  h = jnp.maximum(s * inv + b_ref[...], 0.0)
  o_ref[...] = jnp.dot(h, w_ref[...], preferred_element_type=jnp.float32)


def _act_body(s_ref, d_ref, b_ref, o_ref):
  s = s_ref[0] + s_ref[1]
  deg = d_ref[0] + d_ref[1]
  inv = 1.0 / jnp.maximum(deg[:, 0:1], 1.0)
  o_ref[...] = jnp.maximum(s * inv + b_ref[...], 0.0)


def _final_body(s_ref, d_ref, b_ref, w_ref, o_ref):
  s = s_ref[0] + s_ref[1]
  deg = d_ref[0] + d_ref[1]
  inv = 1.0 / jnp.maximum(deg[:, 0:1], 1.0)
  o_ref[...] = jnp.dot(s * inv, w_ref[...],
                       preferred_element_type=jnp.float32) + b_ref[...]


def _mm(x, w):
  n, d = x.shape
  ko = w.shape[1]
  return pl.pallas_call(
      _mm_body,
      grid=(n // _BR,),
      in_specs=[
          pl.BlockSpec((_BR, d), lambda i: (i, 0)),
          pl.BlockSpec((d, ko), lambda i: (0, 0)),
      ],
      out_specs=pl.BlockSpec((_BR, ko), lambda i: (i, 0)),
      out_shape=jax.ShapeDtypeStruct((n, ko), jnp.float32),
  )(x, w)


def _fuse(sums, degp, b, w):
  _, n, k = sums.shape
  ko = w.shape[1]
  return pl.pallas_call(
      _fuse_body,
      grid=(n // _BR,),
      in_specs=[
          pl.BlockSpec((NC, _BR, k), lambda i: (0, i, 0)),
          pl.BlockSpec((NC, _BR, 16), lambda i: (0, i, 0)),
          pl.BlockSpec((1, k), lambda i: (0, 0)),
          pl.BlockSpec((k, ko), lambda i: (0, 0)),
      ],
      out_specs=pl.BlockSpec((_BR, ko), lambda i: (i, 0)),
      out_shape=jax.ShapeDtypeStruct((n, ko), jnp.float32),
  )(sums, degp, b, w)


def _act(sums, degp, b):
  _, n, k = sums.shape
  return pl.pallas_call(
      _act_body,
      grid=(n // _BR,),
      in_specs=[
          pl.BlockSpec((NC, _BR, k), lambda i: (0, i, 0)),
          pl.BlockSpec((NC, _BR, 16), lambda i: (0, i, 0)),
          pl.BlockSpec((1, k), lambda i: (0, 0)),
      ],
      out_specs=pl.BlockSpec((_BR, k), lambda i: (i, 0)),
      out_shape=jax.ShapeDtypeStruct((n, k), jnp.float32),
  )(sums, degp, b)


def _final(sums, degp, b, w):
  _, n, k = sums.shape
  ko = w.shape[1]
  return pl.pallas_call(
      _final_body,
      grid=(n // _BR,),
      in_specs=[
          pl.BlockSpec((NC, _BR, k), lambda i: (0, i, 0)),
          pl.BlockSpec((NC, _BR, 16), lambda i: (0, i, 0)),
          pl.BlockSpec((1, ko), lambda i: (0, 0)),
          pl.BlockSpec((k, ko), lambda i: (0, 0)),
      ],
      out_specs=pl.BlockSpec((_BR, ko), lambda i: (i, 0)),
      out_shape=jax.ShapeDtypeStruct((n, ko), jnp.float32),
  )(sums, degp, b, w)


def kernel(x, edge_index, W0, b0, W1, b1, W2, b2):
  n, d = x.shape
  e = edge_index.shape[1]
  h = W0.shape[1]
  c = W2.shape[1]
  chunks = e // (NW * CK)

  src3 = edge_index[0]
  dst3 = edge_index[1]

  del c
  agg_d_deg = _make_agg(n, e, d, with_deg=True)
  agg_h = _make_agg(n, e, h, with_deg=False)

  g0 = _mm(x, W0)
  s0, degp = agg_d_deg(g0, src3, dst3)
  g1 = _fuse(s0, degp, b0.reshape(1, -1), W1)
  (s1,) = agg_h(g1, src3, dst3)
  h2 = _act(s1, degp, b1.reshape(1, -1))
  (s2,) = agg_h(h2, src3, dst3)
  return _final(s2, degp, b2.reshape(1, -1), W2)
